# baseline (speedup 1.0000x reference)
import jax
import jax.numpy as jnp
from jax.experimental import pallas as pl
from jax.experimental.pallas import tpu as pltpu

NUM_CLASSES = 100
BATCH = 16384
_GRID = 2
_CHUNKS = 16
_ROWS = BATCH // (_GRID * _CHUNKS)


def _fill_zeros(o_hbm, zbuf, sem):
    g = pl.program_id(0)
    zbuf[...] = jnp.zeros_like(zbuf)
    base = g * (_CHUNKS * _ROWS)
    copies = [
        pltpu.make_async_copy(
            zbuf, o_hbm.at[pl.ds(base + i * _ROWS, _ROWS), :], sem
        )
        for i in range(_CHUNKS)
    ]
    for c in copies:
        c.start()
    for c in copies:
        c.wait()


def kernel(x):
    del x
    return pl.pallas_call(
        _fill_zeros,
        grid=(_GRID,),
        out_specs=pl.BlockSpec(memory_space=pl.ANY),
        out_shape=jax.ShapeDtypeStruct((BATCH, NUM_CLASSES), jnp.float32),
        scratch_shapes=[
            pltpu.MemorySpace.VMEM((_ROWS, NUM_CLASSES), jnp.float32),
            pltpu.SemaphoreType.DMA,
        ],
        compiler_params=pltpu.CompilerParams(
            dimension_semantics=("parallel",),
        ),
    )()


# speedup vs baseline: 1.0536x; 1.0536x over previous
import jax
import jax.numpy as jnp
from jax.experimental import pallas as pl
from jax.experimental.pallas import tpu as pltpu

NUM_CLASSES = 100
BATCH = 16384
_GRID = 1
_CHUNKS = 64
_ROWS = BATCH // (_GRID * _CHUNKS)


def _fill_zeros(o_hbm, zbuf, sem):
    g = pl.program_id(0)
    zbuf[...] = jnp.zeros_like(zbuf)
    base = g * (_CHUNKS * _ROWS)
    copies = [
        pltpu.make_async_copy(
            zbuf, o_hbm.at[pl.ds(base + i * _ROWS, _ROWS), :], sem
        )
        for i in range(_CHUNKS)
    ]
    for c in copies:
        c.start()
    for c in copies:
        c.wait()


def kernel(x):
    del x
    return pl.pallas_call(
        _fill_zeros,
        grid=(_GRID,),
        out_specs=pl.BlockSpec(memory_space=pl.ANY),
        out_shape=jax.ShapeDtypeStruct((BATCH, NUM_CLASSES), jnp.float32),
        scratch_shapes=[
            pltpu.MemorySpace.VMEM((_ROWS, NUM_CLASSES), jnp.float32),
            pltpu.SemaphoreType.DMA,
        ],
        compiler_params=pltpu.CompilerParams(
            dimension_semantics=("parallel",),
        ),
    )()


# speedup vs baseline: 1.0618x; 1.0079x over previous
import jax
import jax.numpy as jnp
from jax.experimental import pallas as pl
from jax.experimental.pallas import tpu as pltpu

NUM_CLASSES = 100
BATCH = 16384
_CHUNKS = 32
_ROWS = BATCH // _CHUNKS


def _fill_zeros(o_hbm, zbuf, sem):
    zbuf[...] = jnp.zeros_like(zbuf)
    copies = [
        pltpu.make_async_copy(zbuf, o_hbm.at[pl.ds(i * _ROWS, _ROWS), :], sem)
        for i in range(_CHUNKS)
    ]
    for c in copies:
        c.start()
    for c in copies:
        c.wait()


def kernel(x):
    del x
    return pl.pallas_call(
        _fill_zeros,
        out_specs=pl.BlockSpec(memory_space=pl.ANY),
        out_shape=jax.ShapeDtypeStruct((BATCH, NUM_CLASSES), jnp.float32),
        scratch_shapes=[
            pltpu.MemorySpace.VMEM((_ROWS, NUM_CLASSES), jnp.float32),
            pltpu.SemaphoreType.DMA,
        ],
    )()
